# baseline (speedup 1.0000x reference)
import functools
import math

import jax
import jax.numpy as jnp
from jax import lax
from jax.experimental import pallas as pl
from jax.experimental.pallas import tpu as pltpu
from jax.experimental.pallas import tpu_sc as plsc

VOCAB = 100000
EMBED_DIM = 128
BATCH = 4096
HIST = 50

NUM_CORES = 2
NUM_SUBCORES = 16
NW = NUM_CORES * NUM_SUBCORES
TOTAL = BATCH * HIST
BPW = TOTAL // NW
CHUNK = 64
NCHUNK = BPW // CHUNK
NBUF = 10
NGRP = NCHUNK // NBUF
LAG = 5
SSIZE = (NCHUNK + 8 - math.gcd(NCHUNK, 8) + 7) // 8 * 8

_mesh = plsc.VectorSubcoreMesh(core_axis_name="c", subcore_axis_name="s")


@functools.partial(
    pl.kernel,
    mesh=_mesh,
    out_type=jax.ShapeDtypeStruct((TOTAL, EMBED_DIM), jnp.float32),
    scratch_types=[
        pltpu.VMEM((SSIZE, CHUNK), jnp.int32),
    ]
    + [pltpu.VMEM((CHUNK, EMBED_DIM), jnp.float32) for _ in range(NBUF)]
    + [pltpu.SemaphoreType.DMA for _ in range(2 * NBUF)],
)
def _embedding_lookup(idx_hbm, table_hbm, out_hbm, idx_v, *bufs_and_sems):
    rows = bufs_and_sems[:NBUF]
    gsem = bufs_and_sems[NBUF:2 * NBUF]
    osem = bufs_and_sems[2 * NBUF:]
    wid = lax.axis_index("s") * NUM_CORES + lax.axis_index("c")
    base = wid * BPW
    roff = lax.rem(wid * NCHUNK, 8)
    start = pl.multiple_of(wid * NCHUNK - roff, 8)
    pltpu.sync_copy(idx_hbm.at[pl.ds(start, SSIZE)], idx_v)

    def gather(c, j):
        return pltpu.make_async_copy(
            table_hbm.at[idx_v.at[roff + c]], rows[j], gsem[j])

    def out_copy(c, j):
        return pltpu.make_async_copy(
            rows[j], out_hbm.at[pl.ds(base + c * CHUNK, CHUNK)], osem[j])

    for c in range(NBUF):
        gather(c, c).start()
        if c >= LAG:
            cd = c - LAG
            gather(cd, cd % NBUF).wait()
            out_copy(cd, cd % NBUF).start()

    def body(t, carry):
        c0 = t * NBUF
        for j in range(NBUF):
            c = c0 + j
            out_copy(c - NBUF, j).wait()
            gather(c, j).start()
            cd = c - LAG
            jd = (j - LAG) % NBUF
            gather(cd, jd).wait()
            out_copy(cd, jd).start()
        return carry

    lax.fori_loop(1, NGRP, body, 0)

    for k in range(LAG):
        cd = NCHUNK - LAG + k
        jd = cd % NBUF
        gather(cd, jd).wait()
        out_copy(cd, jd).start()
    for j in range(NBUF):
        out_copy(NCHUNK - NBUF + j, j).wait()


def kernel(inputs, table):
    idx = inputs.astype(jnp.int32).T.reshape(TOTAL // CHUNK, CHUNK)
    out = _embedding_lookup(idx, table)
    return out.reshape(HIST, BATCH, EMBED_DIM).transpose(1, 0, 2)


# speedup vs baseline: 1.0023x; 1.0023x over previous
import functools
import math

import jax
import jax.numpy as jnp
from jax import lax
from jax.experimental import pallas as pl
from jax.experimental.pallas import tpu as pltpu
from jax.experimental.pallas import tpu_sc as plsc

VOCAB = 100000
EMBED_DIM = 128
BATCH = 4096
HIST = 50

NUM_CORES = 2
NUM_SUBCORES = 16
NW = NUM_CORES * NUM_SUBCORES
TOTAL = BATCH * HIST
BPW = TOTAL // NW
CHUNK = 128
NCHUNK = BPW // CHUNK
NBUF = 5
NGRP = NCHUNK // NBUF
LAG = 2
SSIZE = (NCHUNK + 8 - math.gcd(NCHUNK, 8) + 7) // 8 * 8

_mesh = plsc.VectorSubcoreMesh(core_axis_name="c", subcore_axis_name="s")


@functools.partial(
    pl.kernel,
    mesh=_mesh,
    out_type=jax.ShapeDtypeStruct((TOTAL, EMBED_DIM), jnp.float32),
    scratch_types=[
        pltpu.VMEM((SSIZE, CHUNK), jnp.int32),
    ]
    + [pltpu.VMEM((CHUNK, EMBED_DIM), jnp.float32) for _ in range(NBUF)]
    + [pltpu.SemaphoreType.DMA for _ in range(2 * NBUF)],
)
def _embedding_lookup(idx_hbm, table_hbm, out_hbm, idx_v, *bufs_and_sems):
    rows = bufs_and_sems[:NBUF]
    gsem = bufs_and_sems[NBUF:2 * NBUF]
    osem = bufs_and_sems[2 * NBUF:]
    wid = lax.axis_index("s") * NUM_CORES + lax.axis_index("c")
    base = wid * BPW
    roff = lax.rem(wid * NCHUNK, 8)
    start = pl.multiple_of(wid * NCHUNK - roff, 8)
    pltpu.sync_copy(idx_hbm.at[pl.ds(start, SSIZE)], idx_v)

    def gather(c, j):
        return pltpu.make_async_copy(
            table_hbm.at[idx_v.at[roff + c]], rows[j], gsem[j])

    def out_copy(c, j):
        return pltpu.make_async_copy(
            rows[j], out_hbm.at[pl.ds(base + c * CHUNK, CHUNK)], osem[j])

    for c in range(NBUF):
        gather(c, c).start()
        if c >= LAG:
            cd = c - LAG
            gather(cd, cd % NBUF).wait()
            out_copy(cd, cd % NBUF).start()

    def body(t, carry):
        c0 = t * NBUF
        for j in range(NBUF):
            c = c0 + j
            out_copy(c - NBUF, j).wait()
            gather(c, j).start()
            cd = c - LAG
            jd = (j - LAG) % NBUF
            gather(cd, jd).wait()
            out_copy(cd, jd).start()
        return carry

    lax.fori_loop(1, NGRP, body, 0)

    for k in range(LAG):
        cd = NCHUNK - LAG + k
        jd = cd % NBUF
        gather(cd, jd).wait()
        out_copy(cd, jd).start()
    for j in range(NBUF):
        out_copy(NCHUNK - NBUF + j, j).wait()


def kernel(inputs, table):
    idx = inputs.astype(jnp.int32).T.reshape(TOTAL // CHUNK, CHUNK)
    out = _embedding_lookup(idx, table)
    return out.reshape(HIST, BATCH, EMBED_DIM).transpose(1, 0, 2)


# speedup vs baseline: 1.0192x; 1.0168x over previous
import functools

import jax
import jax.numpy as jnp
from jax import lax
from jax.experimental import pallas as pl
from jax.experimental.pallas import tpu as pltpu
from jax.experimental.pallas import tpu_sc as plsc

VOCAB = 100000
EMBED_DIM = 128
BATCH = 4096
HIST = 50

NUM_CORES = 2
NUM_SUBCORES = 16
NW = NUM_CORES * NUM_SUBCORES
TOTAL = BATCH * HIST
BPW = TOTAL // NW
CHUNK = 128
NCHUNK = HIST
NBUF = 5
NGRP = NCHUNK // NBUF
LAG = 2
_mesh = plsc.VectorSubcoreMesh(core_axis_name="c", subcore_axis_name="s")


@functools.partial(
    pl.kernel,
    mesh=_mesh,
    out_type=jax.ShapeDtypeStruct((TOTAL, EMBED_DIM), jnp.float32),
    scratch_types=[
        pltpu.VMEM((NCHUNK, CHUNK), jnp.int32),
    ]
    + [pltpu.VMEM((CHUNK, EMBED_DIM), jnp.float32) for _ in range(NBUF)]
    + [pltpu.SemaphoreType.DMA for _ in range(2 * NBUF)],
)
def _embedding_lookup(idx_hbm, table_hbm, out_hbm, idx_v, *bufs_and_sems):
    rows = bufs_and_sems[:NBUF]
    gsem = bufs_and_sems[NBUF:2 * NBUF]
    osem = bufs_and_sems[2 * NBUF:]
    wid = lax.axis_index("s") * NUM_CORES + lax.axis_index("c")
    base = wid * CHUNK
    pltpu.sync_copy(idx_hbm.at[:, pl.ds(base, CHUNK)], idx_v)

    def gather(c, j):
        return pltpu.make_async_copy(table_hbm.at[idx_v.at[c]], rows[j], gsem[j])

    def out_copy(c, j):
        return pltpu.make_async_copy(
            rows[j], out_hbm.at[pl.ds(c * BATCH + base, CHUNK)], osem[j])

    for c in range(NBUF):
        gather(c, c).start()
        if c >= LAG:
            cd = c - LAG
            gather(cd, cd % NBUF).wait()
            out_copy(cd, cd % NBUF).start()

    def body(t, carry):
        c0 = t * NBUF
        for j in range(NBUF):
            c = c0 + j
            out_copy(c - NBUF, j).wait()
            gather(c, j).start()
            cd = c - LAG
            jd = (j - LAG) % NBUF
            gather(cd, jd).wait()
            out_copy(cd, jd).start()
        return carry

    lax.fori_loop(1, NGRP, body, 0)

    for k in range(LAG):
        cd = NCHUNK - LAG + k
        jd = cd % NBUF
        gather(cd, jd).wait()
        out_copy(cd, jd).start()
    for j in range(NBUF):
        out_copy(NCHUNK - NBUF + j, j).wait()


def kernel(inputs, table):
    idx = inputs.astype(jnp.int32).T
    out = _embedding_lookup(idx, table)
    return out.reshape(HIST, BATCH, EMBED_DIM).transpose(1, 0, 2)


# speedup vs baseline: 1.0246x; 1.0053x over previous
import functools

import jax
import jax.numpy as jnp
from jax import lax
from jax.experimental import pallas as pl
from jax.experimental.pallas import tpu as pltpu
from jax.experimental.pallas import tpu_sc as plsc

VOCAB = 100000
EMBED_DIM = 128
BATCH = 4096
HIST = 50

NUM_CORES = 2
NUM_SUBCORES = 16
NW = NUM_CORES * NUM_SUBCORES
TOTAL = BATCH * HIST
BPW = TOTAL // NW
CHUNK = 128
NCHUNK = HIST
NBUF = 5
NGRP = NCHUNK // NBUF
LAG = 3
_mesh = plsc.VectorSubcoreMesh(core_axis_name="c", subcore_axis_name="s")


@functools.partial(
    pl.kernel,
    mesh=_mesh,
    out_type=jax.ShapeDtypeStruct((TOTAL, EMBED_DIM), jnp.float32),
    scratch_types=[
        pltpu.VMEM((NCHUNK, CHUNK), jnp.int32),
    ]
    + [pltpu.VMEM((CHUNK, EMBED_DIM), jnp.float32) for _ in range(NBUF)]
    + [pltpu.SemaphoreType.DMA for _ in range(2 * NBUF)],
)
def _embedding_lookup(idx_hbm, table_hbm, out_hbm, idx_v, *bufs_and_sems):
    rows = bufs_and_sems[:NBUF]
    gsem = bufs_and_sems[NBUF:2 * NBUF]
    osem = bufs_and_sems[2 * NBUF:]
    wid = lax.axis_index("s") * NUM_CORES + lax.axis_index("c")
    base = wid * CHUNK
    pltpu.sync_copy(idx_hbm.at[:, pl.ds(base, CHUNK)], idx_v)

    def gather(c, j):
        return pltpu.make_async_copy(table_hbm.at[idx_v.at[c]], rows[j], gsem[j])

    def out_copy(c, j):
        return pltpu.make_async_copy(
            rows[j], out_hbm.at[pl.ds(c * BATCH + base, CHUNK)], osem[j])

    for c in range(NBUF):
        gather(c, c).start()
        if c >= LAG:
            cd = c - LAG
            gather(cd, cd % NBUF).wait()
            out_copy(cd, cd % NBUF).start()

    def body(t, carry):
        c0 = t * NBUF
        for j in range(NBUF):
            c = c0 + j
            out_copy(c - NBUF, j).wait()
            gather(c, j).start()
            cd = c - LAG
            jd = (j - LAG) % NBUF
            gather(cd, jd).wait()
            out_copy(cd, jd).start()
        return carry

    lax.fori_loop(1, NGRP, body, 0)

    for k in range(LAG):
        cd = NCHUNK - LAG + k
        jd = cd % NBUF
        gather(cd, jd).wait()
        out_copy(cd, jd).start()
    for j in range(NBUF):
        out_copy(NCHUNK - NBUF + j, j).wait()


def kernel(inputs, table):
    idx = inputs.astype(jnp.int32).T
    out = _embedding_lookup(idx, table)
    return out.reshape(HIST, BATCH, EMBED_DIM).transpose(1, 0, 2)
